# Initial kernel scaffold; baseline (speedup 1.0000x reference)
import jax
import jax.numpy as jnp
from jax.experimental import pallas as pl


def kernel(input_ids, table, offsets):
    raise NotImplementedError("write your pallas kernel here")


# speedup vs baseline: 1.4981x; 1.4981x over previous
import functools

import jax
import jax.numpy as jnp
from jax import lax
from jax.experimental import pallas as pl
from jax.experimental.pallas import tpu as pltpu
from jax.experimental.pallas import tpu_sc as plsc

NC = 2
NS = 16
L = 16
NW = NC * NS

D = 32
CHUNK = 128
NBUF = 8


def _body(ids_hbm, offpat_hbm, table_hbm, out_hbm, idx_v, off_v, rows_v, sem):
    n_chunks = ids_hbm.shape[1]
    n_groups = n_chunks // NBUF
    wid = lax.axis_index("s") * NC + lax.axis_index("c")

    pltpu.sync_copy(ids_hbm.at[wid], idx_v)
    pltpu.sync_copy(offpat_hbm, off_v)
    off = off_v[...]

    def add_row(j, _):
        for k in range(CHUNK // L):
            sl = pl.ds(k * L, L)
            idx_v[j, sl] = idx_v[j, sl] + off
        return 0

    lax.fori_loop(0, n_chunks, add_row, 0)

    def group(g, _):
        copies = []
        for b in range(NBUF):
            copies.append(
                pltpu.async_copy(
                    table_hbm.at[idx_v.at[g * NBUF + b]], rows_v.at[b], sem
                )
            )
        for c in copies:
            c.wait()
        pltpu.sync_copy(rows_v, out_hbm.at[wid, pl.ds(g * NBUF, NBUF)])
        return 0

    lax.fori_loop(0, n_groups, group, 0)


def kernel(input_ids, table, offsets):
    B, T, H = input_ids.shape
    total = B * T * H
    per_w = total // NW
    n_chunks = per_w // CHUNK

    ids = input_ids.reshape(NW, n_chunks, CHUNK).astype(jnp.int32)
    off_pat = jnp.tile(offsets.astype(jnp.int32), L // H)
    tab = table.astype(jnp.float32)

    run = functools.partial(
        pl.kernel,
        mesh=plsc.VectorSubcoreMesh(core_axis_name="c", subcore_axis_name="s"),
        out_type=jax.ShapeDtypeStruct((NW, n_chunks, CHUNK, D), jnp.float32),
        scratch_types=[
            pltpu.VMEM((n_chunks, CHUNK), jnp.int32),
            pltpu.VMEM((L,), jnp.int32),
            pltpu.VMEM((NBUF, CHUNK, D), jnp.float32),
            pltpu.SemaphoreType.DMA,
        ],
        compiler_params=pltpu.CompilerParams(use_tc_tiling_on_sc=False),
    )(_body)

    out = run(ids, off_pat, tab)
    return out.reshape(B, T, H, D)
